# Initial kernel scaffold; baseline (speedup 1.0000x reference)
import jax
import jax.numpy as jnp
from jax.experimental import pallas as pl


def kernel(x, edge_index_rel0, edge_index_rel1, edge_index_rel2, W1_rel0, b1_rel0, W1_rel1, b1_rel1, W1_rel2, b1_rel2, W2_rel0, b2_rel0, W2_rel1, b2_rel1, W2_rel2, b2_rel2):
    raise NotImplementedError("write your pallas kernel here")


# speedup vs baseline: 5.0982x; 5.0982x over previous
import functools

import jax
import jax.numpy as jnp
from jax import lax
from jax.experimental import pallas as pl
from jax.experimental.pallas import tpu as pltpu
from jax.experimental.pallas import tpu_sc as plsc

N = 10000
E = 200000
F = 128
FH = F // 2
R = 3
NC = 2
NS = 16
NW = NC * NS
CH = 128
NCHUNK = -(-E // (NW * CH))
EP = NW * NCHUNK * CH
NP = 10240
RPT = NP // NS
BLK = 512


def _sc_degrees(src_all, dst_all, zrow):
    mesh = plsc.VectorSubcoreMesh(core_axis_name="c", subcore_axis_name="s")

    @functools.partial(
        pl.kernel,
        out_type=jax.ShapeDtypeStruct((NC, 2 * R, NP), jnp.float32),
        mesh=mesh,
        scratch_types=[
            pltpu.VMEM((NCHUNK, CH), jnp.int32),
            pltpu.VMEM((CH,), jnp.float32),
        ] + [pltpu.VMEM_SHARED((NP,), jnp.float32) for _ in range(2 * R)],
    )
    def body(src_hbm, dst_hbm, zrow_hbm, out_hbm, idx_v, ones_v, *accs):
        c = lax.axis_index("c")
        s = lax.axis_index("s")
        wid = c * NS + s
        for i in range(CH // 16):
            ones_v[pl.ds(i * 16, 16)] = jnp.ones((16,), jnp.float32)
        for k in range(2 * R):
            @pl.when(s == k)
            def _():
                pltpu.sync_copy(zrow_hbm, accs[k])
        plsc.subcore_barrier()
        for r in range(R):
            for arr, base in ((src_hbm, 0), (dst_hbm, R)):
                pltpu.sync_copy(arr.at[r, wid], idx_v)
                acc = accs[base + r]

                def chunk(j, _, acc=acc):
                    pltpu.sync_copy(ones_v, acc.at[idx_v.at[j]], add=True)
                    return 0

                lax.fori_loop(0, NCHUNK, chunk, 0)
        plsc.subcore_barrier()
        for k in range(2 * R):
            @pl.when(s == k)
            def _():
                pltpu.sync_copy(accs[k], out_hbm.at[c, k])

    return body(src_all, dst_all, zrow)


def _sc_propagate(tabs, src_all, dst_all, ztile):
    mesh = plsc.VectorSubcoreMesh(core_axis_name="c", subcore_axis_name="s")

    @functools.partial(
        pl.kernel,
        out_type=jax.ShapeDtypeStruct((2, NC, R, NP, FH), jnp.float32),
        mesh=mesh,
        scratch_types=[
            pltpu.VMEM((NCHUNK, CH), jnp.int32),
            pltpu.VMEM((NCHUNK, CH), jnp.int32),
            pltpu.VMEM((2, CH, FH), jnp.float32),
            pltpu.VMEM((CH, FH), jnp.float32),
            pltpu.VMEM_SHARED((NP, FH), jnp.float32),
            pltpu.SemaphoreType.DMA,
            pltpu.SemaphoreType.DMA,
        ],
        compiler_params=pltpu.CompilerParams(use_tc_tiling_on_sc=False),
    )
    def body(t00, t01, t10, t11, t20, t21, src_hbm, dst_hbm, ztile_hbm,
             out_hbm, sidx, didx, rows, zbuf, acc, sem0, sem1):
        c = lax.axis_index("c")
        s = lax.axis_index("s")
        wid = c * NS + s
        tables = ((t00, t01), (t10, t11), (t20, t21))
        pltpu.sync_copy(ztile_hbm, zbuf)
        row0 = s * RPT
        for r in range(R):
            pltpu.sync_copy(src_hbm.at[r, wid], sidx)
            pltpu.sync_copy(dst_hbm.at[r, wid], didx)
            for h in range(2):
                table = tables[r][h]
                for kk in range(RPT // CH):
                    pltpu.sync_copy(zbuf, acc.at[pl.ds(row0 + kk * CH, CH)])
                plsc.subcore_barrier()
                pltpu.async_copy(table.at[sidx.at[0]], rows.at[0], sem0)

                def pair(i, _, table=table):
                    ja = 2 * i
                    jb = 2 * i + 1
                    pltpu.make_async_copy(table.at[sidx.at[ja]], rows.at[0],
                                          sem0).wait()
                    pltpu.async_copy(table.at[sidx.at[jb]], rows.at[1], sem1)
                    pltpu.sync_copy(rows.at[0], acc.at[didx.at[ja]], add=True)
                    pltpu.make_async_copy(table.at[sidx.at[jb]], rows.at[1],
                                          sem1).wait()
                    pltpu.async_copy(table.at[sidx.at[jb + 1]], rows.at[0], sem0)
                    pltpu.sync_copy(rows.at[1], acc.at[didx.at[jb]], add=True)
                    return 0

                lax.fori_loop(0, (NCHUNK - 1) // 2, pair, 0)
                pltpu.make_async_copy(table.at[sidx.at[NCHUNK - 1]], rows.at[0],
                                      sem0).wait()
                pltpu.sync_copy(rows.at[0], acc.at[didx.at[NCHUNK - 1]],
                                add=True)
                plsc.subcore_barrier()
                pltpu.sync_copy(acc.at[pl.ds(row0, RPT)],
                                out_hbm.at[h, c, r, pl.ds(row0, RPT)])

    return body(*tabs, src_all, dst_all, ztile)


def _norm_from_deg(degsum):
    return jnp.where(degsum > 0, lax.rsqrt(jnp.maximum(degsum, 1.0)), 0.0)


def _tc_norms_scale(deg_part, x_p):
    def body(deg_ref, x_ref, ns_ref, nd_ref, *touts):
        deg = deg_ref[...]
        norm = _norm_from_deg(deg[0] + deg[1])
        ns_ref[...] = norm[:R]
        nd_ref[...] = norm[R:]
        xv = x_ref[...]
        for r in range(R):
            scaled = xv * norm[r][:, None]
            touts[2 * r][...] = scaled[:, :FH]
            touts[2 * r + 1][...] = scaled[:, FH:]

    grid = (NP // BLK,)
    return pl.pallas_call(
        body,
        grid=grid,
        in_specs=[
            pl.BlockSpec((NC, 2 * R, BLK), lambda i: (0, 0, i)),
            pl.BlockSpec((BLK, F), lambda i: (i, 0)),
        ],
        out_specs=[
            pl.BlockSpec((R, BLK), lambda i: (0, i)),
            pl.BlockSpec((R, BLK), lambda i: (0, i)),
        ] + [pl.BlockSpec((BLK, FH), lambda i: (i, 0)) for _ in range(2 * R)],
        out_shape=[
            jax.ShapeDtypeStruct((R, NP), jnp.float32),
            jax.ShapeDtypeStruct((R, NP), jnp.float32),
        ] + [jax.ShapeDtypeStruct((NP, FH), jnp.float32) for _ in range(2 * R)],
    )(deg_part, x_p)


def _agg_rows(p, nd, w_ref, b):
    t = jnp.zeros((BLK, F), jnp.float32)
    for r in range(R):
        aggr = jnp.concatenate(
            [p[0, 0, r] + p[0, 1, r], p[1, 0, r] + p[1, 1, r]], axis=1)
        t = t + nd[r][:, None] * jnp.dot(
            aggr, w_ref[r], precision=lax.Precision.HIGHEST,
            preferred_element_type=jnp.float32)
    return t + (b[0] + b[1] + b[2])[None, :]


def _tc_mid(part, Ws, bs, nd, ns):
    def body(p_ref, w_ref, b_ref, nd_ref, ns_ref, *houts):
        p = p_ref[...]
        h = jnp.maximum(_agg_rows(p, nd_ref[...], w_ref, b_ref[...]), 0.0)
        ns = ns_ref[...]
        for r in range(R):
            scaled = h * ns[r][:, None]
            houts[2 * r][...] = scaled[:, :FH]
            houts[2 * r + 1][...] = scaled[:, FH:]

    grid = (NP // BLK,)
    return pl.pallas_call(
        body,
        grid=grid,
        in_specs=[
            pl.BlockSpec((2, NC, R, BLK, FH), lambda i: (0, 0, 0, i, 0)),
            pl.BlockSpec((R, F, F), lambda i: (0, 0, 0)),
            pl.BlockSpec((R, F), lambda i: (0, 0)),
            pl.BlockSpec((R, BLK), lambda i: (0, i)),
            pl.BlockSpec((R, BLK), lambda i: (0, i)),
        ],
        out_specs=[pl.BlockSpec((BLK, FH), lambda i: (i, 0))
                   for _ in range(2 * R)],
        out_shape=[jax.ShapeDtypeStruct((NP, FH), jnp.float32)
                   for _ in range(2 * R)],
    )(part, Ws, bs, nd, ns)


def _tc_final(part, Ws, bs, nd):
    def body(p_ref, w_ref, b_ref, nd_ref, o_ref):
        o_ref[...] = _agg_rows(p_ref[...], nd_ref[...], w_ref, b_ref[...])

    grid = (NP // BLK,)
    return pl.pallas_call(
        body,
        grid=grid,
        in_specs=[
            pl.BlockSpec((2, NC, R, BLK, FH), lambda i: (0, 0, 0, i, 0)),
            pl.BlockSpec((R, F, F), lambda i: (0, 0, 0)),
            pl.BlockSpec((R, F), lambda i: (0, 0)),
            pl.BlockSpec((R, BLK), lambda i: (0, i)),
        ],
        out_specs=pl.BlockSpec((BLK, F), lambda i: (i, 0)),
        out_shape=jax.ShapeDtypeStruct((NP, F), jnp.float32),
    )(part, Ws, bs, nd)


def _prep_edges(ei):
    pad = EP - E
    src = jnp.concatenate([ei[0], jnp.full((pad,), N, jnp.int32)])
    dst = jnp.concatenate([ei[1], jnp.full((pad,), N, jnp.int32)])
    return src.reshape(NW, NCHUNK, CH), dst.reshape(NW, NCHUNK, CH)


def kernel(x, edge_index_rel0, edge_index_rel1, edge_index_rel2,
           W1_rel0, b1_rel0, W1_rel1, b1_rel1, W1_rel2, b1_rel2,
           W2_rel0, b2_rel0, W2_rel1, b2_rel1, W2_rel2, b2_rel2):
    s0, d0 = _prep_edges(edge_index_rel0)
    s1, d1 = _prep_edges(edge_index_rel1)
    s2, d2 = _prep_edges(edge_index_rel2)
    src_all = jnp.stack([s0, s1, s2])
    dst_all = jnp.stack([d0, d1, d2])

    x_p = jnp.zeros((NP, F), jnp.float32).at[:N].set(x)
    zrow = jnp.zeros((NP,), jnp.float32)
    ztile = jnp.zeros((CH, FH), jnp.float32)
    W1s = jnp.stack([W1_rel0, W1_rel1, W1_rel2])
    b1s = jnp.stack([b1_rel0, b1_rel1, b1_rel2])
    W2s = jnp.stack([W2_rel0, W2_rel1, W2_rel2])
    b2s = jnp.stack([b2_rel0, b2_rel1, b2_rel2])

    deg_part = _sc_degrees(src_all, dst_all, zrow)
    ns, nd, *xtabs = _tc_norms_scale(deg_part, x_p)
    part1 = _sc_propagate(xtabs, src_all, dst_all, ztile)
    htabs = _tc_mid(part1, W1s, b1s, nd, ns)
    part2 = _sc_propagate(htabs, src_all, dst_all, ztile)
    out = _tc_final(part2, W2s, b2s, nd)
    return out[:N]


# speedup vs baseline: 7.8445x; 1.5387x over previous
import functools

import jax
import jax.numpy as jnp
from jax import lax
from jax.experimental import pallas as pl
from jax.experimental.pallas import tpu as pltpu
from jax.experimental.pallas import tpu_sc as plsc

N = 10000
E = 200000
F = 128
FH = F // 2
R = 3
NC = 2
NS = 16
NW = NC * NS
CH = 128
RING = 7
NG = 7
NCHUNK = RING * NG
EP = NW * NCHUNK * CH
NP = 10240
RPT = NP // NS
BLK = 512


def _sc_degrees(src_all, dst_all, zrow):
    mesh = plsc.VectorSubcoreMesh(core_axis_name="c", subcore_axis_name="s")

    @functools.partial(
        pl.kernel,
        out_type=jax.ShapeDtypeStruct((NC, 2 * R, NP), jnp.float32),
        mesh=mesh,
        scratch_types=[
            pltpu.VMEM((NCHUNK, CH), jnp.int32),
            pltpu.VMEM((CH,), jnp.float32),
        ] + [pltpu.VMEM_SHARED((NP,), jnp.float32) for _ in range(2 * R)],
    )
    def body(src_hbm, dst_hbm, zrow_hbm, out_hbm, idx_v, ones_v, *accs):
        c = lax.axis_index("c")
        s = lax.axis_index("s")
        wid = c * NS + s
        for i in range(CH // 16):
            ones_v[pl.ds(i * 16, 16)] = jnp.ones((16,), jnp.float32)
        for k in range(2 * R):
            @pl.when(s == k)
            def _():
                pltpu.sync_copy(zrow_hbm, accs[k])
        plsc.subcore_barrier()
        for r in range(R):
            for arr, base in ((src_hbm, 0), (dst_hbm, R)):
                pltpu.sync_copy(arr.at[r, wid], idx_v)
                acc = accs[base + r]

                def chunk(j, _, acc=acc):
                    pltpu.sync_copy(ones_v, acc.at[idx_v.at[j]], add=True)
                    return 0

                lax.fori_loop(0, NCHUNK, chunk, 0)
        plsc.subcore_barrier()
        for k in range(2 * R):
            @pl.when(s == k)
            def _():
                pltpu.sync_copy(accs[k], out_hbm.at[c, k])

    return body(src_all, dst_all, zrow)


def _sc_propagate(tabs, src_all, dst_all, ztile):
    mesh = plsc.VectorSubcoreMesh(core_axis_name="c", subcore_axis_name="s")

    @functools.partial(
        pl.kernel,
        out_type=jax.ShapeDtypeStruct((2, NC, R, NP, FH), jnp.float32),
        mesh=mesh,
        scratch_types=[
            pltpu.VMEM((NCHUNK, CH), jnp.int32),
            pltpu.VMEM((NCHUNK, CH), jnp.int32),
            pltpu.VMEM((RING, CH, FH), jnp.float32),
            pltpu.VMEM((CH, FH), jnp.float32),
            pltpu.VMEM_SHARED((NP, FH), jnp.float32),
        ] + [pltpu.SemaphoreType.DMA for _ in range(2 * RING)],
        compiler_params=pltpu.CompilerParams(use_tc_tiling_on_sc=False),
    )
    def body(t00, t01, t10, t11, t20, t21, src_hbm, dst_hbm, ztile_hbm,
             out_hbm, sidx, didx, rows, zbuf, acc, *sems):
        gsem = sems[:RING]
        ssem = sems[RING:]
        c = lax.axis_index("c")
        s = lax.axis_index("s")
        wid = c * NS + s
        tables = ((t00, t01), (t10, t11), (t20, t21))
        pltpu.sync_copy(ztile_hbm, zbuf)
        row0 = s * RPT

        def gather(table, j, b):
            pltpu.async_copy(table.at[sidx.at[j]], rows.at[b], gsem[b])

        def wait_gather(table, b):
            pltpu.make_async_copy(table.at[sidx.at[0]], rows.at[b],
                                  gsem[b]).wait()

        def scatter(j, b):
            pltpu.async_copy(rows.at[b], acc.at[didx.at[j]], ssem[b], add=True)

        def wait_scatter(b):
            pltpu.make_async_copy(rows.at[b], acc.at[didx.at[0]],
                                  ssem[b]).wait()

        for r in range(R):
            pltpu.sync_copy(src_hbm.at[r, wid], sidx)
            pltpu.sync_copy(dst_hbm.at[r, wid], didx)
            for h in range(2):
                table = tables[r][h]
                for kk in range(RPT // CH):
                    pltpu.sync_copy(zbuf, acc.at[pl.ds(row0 + kk * CH, CH)])
                plsc.subcore_barrier()
                for b in range(RING):
                    gather(table, b, b)

                def group(g, _, table=table):
                    j0 = g * RING
                    for b in range(RING):
                        wait_gather(table, b)
                        scatter(j0 + b, b)
                    for b in range(RING):
                        wait_scatter(b)
                        gather(table, j0 + RING + b, b)
                    return 0

                lax.fori_loop(0, NG - 1, group, 0)
                j0 = (NG - 1) * RING
                for b in range(RING):
                    wait_gather(table, b)
                    scatter(j0 + b, b)
                for b in range(RING):
                    wait_scatter(b)
                plsc.subcore_barrier()
                pltpu.sync_copy(acc.at[pl.ds(row0, RPT)],
                                out_hbm.at[h, c, r, pl.ds(row0, RPT)])

    return body(*tabs, src_all, dst_all, ztile)


def _norm_from_deg(degsum):
    return jnp.where(degsum > 0, lax.rsqrt(jnp.maximum(degsum, 1.0)), 0.0)


def _tc_norms_scale(deg_part, x_p):
    def body(deg_ref, x_ref, ns_ref, nd_ref, *touts):
        deg = deg_ref[...]
        norm = _norm_from_deg(deg[0] + deg[1])
        ns_ref[...] = norm[:R]
        nd_ref[...] = norm[R:]
        xv = x_ref[...]
        for r in range(R):
            scaled = xv * norm[r][:, None]
            touts[2 * r][...] = scaled[:, :FH]
            touts[2 * r + 1][...] = scaled[:, FH:]

    grid = (NP // BLK,)
    return pl.pallas_call(
        body,
        grid=grid,
        in_specs=[
            pl.BlockSpec((NC, 2 * R, BLK), lambda i: (0, 0, i)),
            pl.BlockSpec((BLK, F), lambda i: (i, 0)),
        ],
        out_specs=[
            pl.BlockSpec((R, BLK), lambda i: (0, i)),
            pl.BlockSpec((R, BLK), lambda i: (0, i)),
        ] + [pl.BlockSpec((BLK, FH), lambda i: (i, 0)) for _ in range(2 * R)],
        out_shape=[
            jax.ShapeDtypeStruct((R, NP), jnp.float32),
            jax.ShapeDtypeStruct((R, NP), jnp.float32),
        ] + [jax.ShapeDtypeStruct((NP, FH), jnp.float32) for _ in range(2 * R)],
    )(deg_part, x_p)


def _agg_rows(p, nd, w_ref, b):
    t = jnp.zeros((BLK, F), jnp.float32)
    for r in range(R):
        aggr = jnp.concatenate(
            [p[0, 0, r] + p[0, 1, r], p[1, 0, r] + p[1, 1, r]], axis=1)
        t = t + nd[r][:, None] * jnp.dot(
            aggr, w_ref[r], precision=lax.Precision.HIGHEST,
            preferred_element_type=jnp.float32)
    return t + (b[0] + b[1] + b[2])[None, :]


def _tc_mid(part, Ws, bs, nd, ns):
    def body(p_ref, w_ref, b_ref, nd_ref, ns_ref, *houts):
        p = p_ref[...]
        h = jnp.maximum(_agg_rows(p, nd_ref[...], w_ref, b_ref[...]), 0.0)
        ns = ns_ref[...]
        for r in range(R):
            scaled = h * ns[r][:, None]
            houts[2 * r][...] = scaled[:, :FH]
            houts[2 * r + 1][...] = scaled[:, FH:]

    grid = (NP // BLK,)
    return pl.pallas_call(
        body,
        grid=grid,
        in_specs=[
            pl.BlockSpec((2, NC, R, BLK, FH), lambda i: (0, 0, 0, i, 0)),
            pl.BlockSpec((R, F, F), lambda i: (0, 0, 0)),
            pl.BlockSpec((R, F), lambda i: (0, 0)),
            pl.BlockSpec((R, BLK), lambda i: (0, i)),
            pl.BlockSpec((R, BLK), lambda i: (0, i)),
        ],
        out_specs=[pl.BlockSpec((BLK, FH), lambda i: (i, 0))
                   for _ in range(2 * R)],
        out_shape=[jax.ShapeDtypeStruct((NP, FH), jnp.float32)
                   for _ in range(2 * R)],
    )(part, Ws, bs, nd, ns)


def _tc_final(part, Ws, bs, nd):
    def body(p_ref, w_ref, b_ref, nd_ref, o_ref):
        o_ref[...] = _agg_rows(p_ref[...], nd_ref[...], w_ref, b_ref[...])

    grid = (NP // BLK,)
    return pl.pallas_call(
        body,
        grid=grid,
        in_specs=[
            pl.BlockSpec((2, NC, R, BLK, FH), lambda i: (0, 0, 0, i, 0)),
            pl.BlockSpec((R, F, F), lambda i: (0, 0, 0)),
            pl.BlockSpec((R, F), lambda i: (0, 0)),
            pl.BlockSpec((R, BLK), lambda i: (0, i)),
        ],
        out_specs=pl.BlockSpec((BLK, F), lambda i: (i, 0)),
        out_shape=jax.ShapeDtypeStruct((NP, F), jnp.float32),
    )(part, Ws, bs, nd)


def _prep_edges(ei):
    pad = EP - E
    fill = N + (jnp.arange(pad, dtype=jnp.int32) % (NP - N))
    src = jnp.concatenate([ei[0], fill])
    dst = jnp.concatenate([ei[1], fill])
    return src.reshape(NW, NCHUNK, CH), dst.reshape(NW, NCHUNK, CH)


def kernel(x, edge_index_rel0, edge_index_rel1, edge_index_rel2,
           W1_rel0, b1_rel0, W1_rel1, b1_rel1, W1_rel2, b1_rel2,
           W2_rel0, b2_rel0, W2_rel1, b2_rel1, W2_rel2, b2_rel2):
    s0, d0 = _prep_edges(edge_index_rel0)
    s1, d1 = _prep_edges(edge_index_rel1)
    s2, d2 = _prep_edges(edge_index_rel2)
    src_all = jnp.stack([s0, s1, s2])
    dst_all = jnp.stack([d0, d1, d2])

    x_p = jnp.zeros((NP, F), jnp.float32).at[:N].set(x)
    zrow = jnp.zeros((NP,), jnp.float32)
    ztile = jnp.zeros((CH, FH), jnp.float32)
    W1s = jnp.stack([W1_rel0, W1_rel1, W1_rel2])
    b1s = jnp.stack([b1_rel0, b1_rel1, b1_rel2])
    W2s = jnp.stack([W2_rel0, W2_rel1, W2_rel2])
    b2s = jnp.stack([b2_rel0, b2_rel1, b2_rel2])

    deg_part = _sc_degrees(src_all, dst_all, zrow)
    ns, nd, *xtabs = _tc_norms_scale(deg_part, x_p)
    part1 = _sc_propagate(xtabs, src_all, dst_all, ztile)
    htabs = _tc_mid(part1, W1s, b1s, nd, ns)
    part2 = _sc_propagate(htabs, src_all, dst_all, ztile)
    out = _tc_final(part2, W2s, b2s, nd)
    return out[:N]


# speedup vs baseline: 8.3477x; 1.0641x over previous
import functools

import jax
import jax.numpy as jnp
from jax import lax
from jax.experimental import pallas as pl
from jax.experimental.pallas import tpu as pltpu
from jax.experimental.pallas import tpu_sc as plsc

N = 10000
E = 200000
F = 128
FH = F // 2
R = 3
NC = 2
NS = 16
NW = NC * NS
CH = 128
RING = 7
NG = 14
NCHUNK = RING * NG
NCD = NCHUNK // 2
EP = NS * NCHUNK * CH
NP = 10240
RPT = NP // NS
BLK = 512


def _sc_degrees(src_all, dst_all, zrow):
    mesh = plsc.VectorSubcoreMesh(core_axis_name="c", subcore_axis_name="s")

    @functools.partial(
        pl.kernel,
        out_type=jax.ShapeDtypeStruct((NC, 2 * R, NP), jnp.float32),
        mesh=mesh,
        scratch_types=[
            pltpu.VMEM((NCD, CH), jnp.int32),
            pltpu.VMEM((CH,), jnp.float32),
        ] + [pltpu.VMEM_SHARED((NP,), jnp.float32) for _ in range(2 * R)],
        compiler_params=pltpu.CompilerParams(use_tc_tiling_on_sc=False),
    )
    def body(src_hbm, dst_hbm, zrow_hbm, out_hbm, idx_v, ones_v, *accs):
        c = lax.axis_index("c")
        s = lax.axis_index("s")
        for i in range(CH // 16):
            ones_v[pl.ds(i * 16, 16)] = jnp.ones((16,), jnp.float32)
        for k in range(2 * R):
            @pl.when(s == k)
            def _():
                pltpu.sync_copy(zrow_hbm, accs[k])
        plsc.subcore_barrier()
        for r in range(R):
            for arr, base in ((src_hbm, 0), (dst_hbm, R)):
                pltpu.sync_copy(arr.at[r, s, pl.ds(c * NCD, NCD)], idx_v)
                acc = accs[base + r]

                def chunk(j, _, acc=acc):
                    pltpu.sync_copy(ones_v, acc.at[idx_v.at[j]], add=True)
                    return 0

                lax.fori_loop(0, NCD, chunk, 0)
        plsc.subcore_barrier()
        for k in range(2 * R):
            @pl.when(s == k)
            def _():
                pltpu.sync_copy(accs[k], out_hbm.at[c, k])

    return body(src_all, dst_all, zrow)


def _sc_propagate(tabs, src_all, dst_all, ztile):
    mesh = plsc.VectorSubcoreMesh(core_axis_name="c", subcore_axis_name="s")

    @functools.partial(
        pl.kernel,
        out_type=jax.ShapeDtypeStruct((NC, R, NP, FH), jnp.float32),
        mesh=mesh,
        scratch_types=[
            pltpu.VMEM((NCHUNK, CH), jnp.int32),
            pltpu.VMEM((NCHUNK, CH), jnp.int32),
            pltpu.VMEM((RING, CH, FH), jnp.float32),
            pltpu.VMEM_SHARED((NP, FH), jnp.float32),
        ] + [pltpu.SemaphoreType.DMA for _ in range(2 * RING)],
        compiler_params=pltpu.CompilerParams(use_tc_tiling_on_sc=False),
    )
    def body(t00, t01, t10, t11, t20, t21, src_hbm, dst_hbm, ztile_hbm,
             out_hbm, sidx, didx, rows, acc, *sems):
        gsem = sems[:RING]
        ssem = sems[RING:]
        c = lax.axis_index("c")
        s = lax.axis_index("s")
        tables = ((t00, t01), (t10, t11), (t20, t21))
        row0 = s * RPT

        def gather(table, j, b):
            pltpu.async_copy(table.at[sidx.at[j]], rows.at[b], gsem[b])

        def wait_gather(table, b):
            pltpu.make_async_copy(table.at[sidx.at[0]], rows.at[b],
                                  gsem[b]).wait()

        def scatter(j, b):
            pltpu.async_copy(rows.at[b], acc.at[didx.at[j]], ssem[b], add=True)

        def wait_scatter(b):
            pltpu.make_async_copy(rows.at[b], acc.at[didx.at[0]],
                                  ssem[b]).wait()

        for r in range(R):
            pltpu.sync_copy(src_hbm.at[r, s], sidx)
            pltpu.sync_copy(dst_hbm.at[r, s], didx)
            for kk in range(RPT // CH):
                pltpu.sync_copy(ztile_hbm, acc.at[pl.ds(row0 + kk * CH, CH)])
            plsc.subcore_barrier()
            for cc in range(NC):
                @pl.when(c == cc)
                def _(table=tables[r][cc]):
                    for b in range(RING):
                        gather(table, b, b)

                    def group(g, _, table=table):
                        j0 = g * RING
                        for b in range(RING):
                            wait_gather(table, b)
                            scatter(j0 + b, b)
                        for b in range(RING):
                            wait_scatter(b)
                            gather(table, j0 + RING + b, b)
                        return 0

                    lax.fori_loop(0, NG - 1, group, 0)
                    j0 = (NG - 1) * RING
                    for b in range(RING):
                        wait_gather(table, b)
                        scatter(j0 + b, b)
                    for b in range(RING):
                        wait_scatter(b)
            plsc.subcore_barrier()
            pltpu.sync_copy(acc.at[pl.ds(row0, RPT)],
                            out_hbm.at[c, r, pl.ds(row0, RPT)])

    return body(*tabs, src_all, dst_all, ztile)


def _norm_from_deg(degsum):
    return jnp.where(degsum > 0, lax.rsqrt(jnp.maximum(degsum, 1.0)), 0.0)


def _tc_norms_scale(deg_part, x_p):
    def body(deg_ref, x_ref, ns_ref, nd_ref, *touts):
        deg = deg_ref[...]
        norm = _norm_from_deg(deg[0] + deg[1])
        ns_ref[...] = norm[:R]
        nd_ref[...] = norm[R:]
        xv = x_ref[...]
        for r in range(R):
            scaled = xv * norm[r][:, None]
            touts[2 * r][...] = scaled[:, :FH]
            touts[2 * r + 1][...] = scaled[:, FH:]

    grid = (NP // BLK,)
    return pl.pallas_call(
        body,
        grid=grid,
        in_specs=[
            pl.BlockSpec((NC, 2 * R, BLK), lambda i: (0, 0, i)),
            pl.BlockSpec((BLK, F), lambda i: (i, 0)),
        ],
        out_specs=[
            pl.BlockSpec((R, BLK), lambda i: (0, i)),
            pl.BlockSpec((R, BLK), lambda i: (0, i)),
        ] + [pl.BlockSpec((BLK, FH), lambda i: (i, 0)) for _ in range(2 * R)],
        out_shape=[
            jax.ShapeDtypeStruct((R, NP), jnp.float32),
            jax.ShapeDtypeStruct((R, NP), jnp.float32),
        ] + [jax.ShapeDtypeStruct((NP, FH), jnp.float32) for _ in range(2 * R)],
    )(deg_part, x_p)


def _agg_rows(p, nd, w_ref, b):
    t = jnp.zeros((BLK, F), jnp.float32)
    for r in range(R):
        aggr = jnp.concatenate([p[0, r], p[1, r]], axis=1)
        t = t + nd[r][:, None] * jnp.dot(
            aggr, w_ref[r], precision=lax.Precision.HIGHEST,
            preferred_element_type=jnp.float32)
    return t + (b[0] + b[1] + b[2])[None, :]


def _tc_mid(part, Ws, bs, nd, ns):
    def body(p_ref, w_ref, b_ref, nd_ref, ns_ref, *houts):
        p = p_ref[...]
        h = jnp.maximum(_agg_rows(p, nd_ref[...], w_ref, b_ref[...]), 0.0)
        ns = ns_ref[...]
        for r in range(R):
            scaled = h * ns[r][:, None]
            houts[2 * r][...] = scaled[:, :FH]
            houts[2 * r + 1][...] = scaled[:, FH:]

    grid = (NP // BLK,)
    return pl.pallas_call(
        body,
        grid=grid,
        in_specs=[
            pl.BlockSpec((NC, R, BLK, FH), lambda i: (0, 0, i, 0)),
            pl.BlockSpec((R, F, F), lambda i: (0, 0, 0)),
            pl.BlockSpec((R, F), lambda i: (0, 0)),
            pl.BlockSpec((R, BLK), lambda i: (0, i)),
            pl.BlockSpec((R, BLK), lambda i: (0, i)),
        ],
        out_specs=[pl.BlockSpec((BLK, FH), lambda i: (i, 0))
                   for _ in range(2 * R)],
        out_shape=[jax.ShapeDtypeStruct((NP, FH), jnp.float32)
                   for _ in range(2 * R)],
    )(part, Ws, bs, nd, ns)


def _tc_final(part, Ws, bs, nd):
    def body(p_ref, w_ref, b_ref, nd_ref, o_ref):
        o_ref[...] = _agg_rows(p_ref[...], nd_ref[...], w_ref, b_ref[...])

    grid = (NP // BLK,)
    return pl.pallas_call(
        body,
        grid=grid,
        in_specs=[
            pl.BlockSpec((NC, R, BLK, FH), lambda i: (0, 0, i, 0)),
            pl.BlockSpec((R, F, F), lambda i: (0, 0, 0)),
            pl.BlockSpec((R, F), lambda i: (0, 0)),
            pl.BlockSpec((R, BLK), lambda i: (0, i)),
        ],
        out_specs=pl.BlockSpec((BLK, F), lambda i: (i, 0)),
        out_shape=jax.ShapeDtypeStruct((NP, F), jnp.float32),
    )(part, Ws, bs, nd)


def _prep_edges(ei):
    pad = EP - E
    fill = N + (jnp.arange(pad, dtype=jnp.int32) % (NP - N))
    src = jnp.concatenate([ei[0], fill])
    dst = jnp.concatenate([ei[1], fill])
    return src.reshape(NS, NCHUNK, CH), dst.reshape(NS, NCHUNK, CH)


def kernel(x, edge_index_rel0, edge_index_rel1, edge_index_rel2,
           W1_rel0, b1_rel0, W1_rel1, b1_rel1, W1_rel2, b1_rel2,
           W2_rel0, b2_rel0, W2_rel1, b2_rel1, W2_rel2, b2_rel2):
    s0, d0 = _prep_edges(edge_index_rel0)
    s1, d1 = _prep_edges(edge_index_rel1)
    s2, d2 = _prep_edges(edge_index_rel2)
    src_all = jnp.stack([s0, s1, s2])
    dst_all = jnp.stack([d0, d1, d2])

    x_p = jnp.zeros((NP, F), jnp.float32).at[:N].set(x)
    zrow = jnp.zeros((NP,), jnp.float32)
    ztile = jnp.zeros((CH, FH), jnp.float32)
    W1s = jnp.stack([W1_rel0, W1_rel1, W1_rel2])
    b1s = jnp.stack([b1_rel0, b1_rel1, b1_rel2])
    W2s = jnp.stack([W2_rel0, W2_rel1, W2_rel2])
    b2s = jnp.stack([b2_rel0, b2_rel1, b2_rel2])

    deg_part = _sc_degrees(src_all, dst_all, zrow)
    ns, nd, *xtabs = _tc_norms_scale(deg_part, x_p)
    part1 = _sc_propagate(xtabs, src_all, dst_all, ztile)
    htabs = _tc_mid(part1, W1s, b1s, nd, ns)
    part2 = _sc_propagate(htabs, src_all, dst_all, ztile)
    out = _tc_final(part2, W2s, b2s, nd)
    return out[:N]


# speedup vs baseline: 10.4730x; 1.2546x over previous
import functools

import jax
import jax.numpy as jnp
from jax import lax
from jax.experimental import pallas as pl
from jax.experimental.pallas import tpu as pltpu
from jax.experimental.pallas import tpu_sc as plsc

N = 10000
E = 200000
F = 128
FH = F // 2
R = 3
NC = 2
NS = 16
NW = NC * NS
CH = 128
RING = 7
NG = 14
NCHUNK = RING * NG
NCD = NCHUNK // 2
NGC = NCD // RING
EP = NS * NCHUNK * CH
NP = 10240
RPT = NP // NS
BLK = 512


def _sc_degrees(src_all, dst_all, zrow):
    mesh = plsc.VectorSubcoreMesh(core_axis_name="c", subcore_axis_name="s")

    @functools.partial(
        pl.kernel,
        out_type=jax.ShapeDtypeStruct((NC, 2 * R, NP), jnp.float32),
        mesh=mesh,
        scratch_types=[
            pltpu.VMEM((NCD, CH), jnp.int32),
            pltpu.VMEM((CH,), jnp.float32),
        ] + [pltpu.VMEM_SHARED((NP,), jnp.float32) for _ in range(2 * R)],
        compiler_params=pltpu.CompilerParams(use_tc_tiling_on_sc=False),
    )
    def body(src_hbm, dst_hbm, zrow_hbm, out_hbm, idx_v, ones_v, *accs):
        c = lax.axis_index("c")
        s = lax.axis_index("s")
        for i in range(CH // 16):
            ones_v[pl.ds(i * 16, 16)] = jnp.ones((16,), jnp.float32)
        for k in range(2 * R):
            @pl.when(s == k)
            def _():
                pltpu.sync_copy(zrow_hbm, accs[k])
        plsc.subcore_barrier()
        for r in range(R):
            for arr, base in ((src_hbm, 0), (dst_hbm, R)):
                pltpu.sync_copy(arr.at[r, s, pl.ds(c * NCD, NCD)], idx_v)
                acc = accs[base + r]

                def chunk(j, _, acc=acc):
                    pltpu.sync_copy(ones_v, acc.at[idx_v.at[j]], add=True)
                    return 0

                lax.fori_loop(0, NCD, chunk, 0)
        plsc.subcore_barrier()
        for k in range(2 * R):
            @pl.when(s == k)
            def _():
                pltpu.sync_copy(accs[k], out_hbm.at[c, k])

    return body(src_all, dst_all, zrow)


def _sc_propagate(tabs, src_all, dst_all, ztile):
    mesh = plsc.VectorSubcoreMesh(core_axis_name="c", subcore_axis_name="s")

    @functools.partial(
        pl.kernel,
        out_type=jax.ShapeDtypeStruct((NC, R, NP, F), jnp.bfloat16),
        mesh=mesh,
        scratch_types=[
            pltpu.VMEM((NCD, CH), jnp.int32),
            pltpu.VMEM((NCD, CH), jnp.int32),
            pltpu.VMEM((CH, F), jnp.bfloat16),
            pltpu.VMEM((RING, CH, F), jnp.bfloat16),
            pltpu.VMEM_SHARED((NP, F), jnp.bfloat16),
        ] + [pltpu.SemaphoreType.DMA for _ in range(2 * RING)],
        compiler_params=pltpu.CompilerParams(use_tc_tiling_on_sc=False),
    )
    def body(t0, t1, t2, src_hbm, dst_hbm, ztile_hbm,
             out_hbm, sidx, didx, zt, rows, acc, *sems):
        gsem = sems[:RING]
        ssem = sems[RING:]
        c = lax.axis_index("c")
        s = lax.axis_index("s")
        tables = (t0, t1, t2)
        row0 = s * RPT

        def gather(table, j, b):
            pltpu.async_copy(table.at[sidx.at[j]], rows.at[b], gsem[b])

        def wait_gather(table, b):
            pltpu.make_async_copy(table.at[sidx.at[0]], rows.at[b],
                                  gsem[b]).wait()

        def scatter(j, b):
            pltpu.async_copy(rows.at[b], acc.at[didx.at[j]], ssem[b], add=True)

        def wait_scatter(b):
            pltpu.make_async_copy(rows.at[b], acc.at[didx.at[0]],
                                  ssem[b]).wait()

        pltpu.sync_copy(ztile_hbm, zt)
        for r in range(R):
            table = tables[r]
            pltpu.sync_copy(src_hbm.at[r, s, pl.ds(c * NCD, NCD)], sidx)
            pltpu.sync_copy(dst_hbm.at[r, s, pl.ds(c * NCD, NCD)], didx)
            for kk in range(RPT // CH):
                pltpu.sync_copy(zt, acc.at[pl.ds(row0 + kk * CH, CH)])
            plsc.subcore_barrier()
            for b in range(RING):
                gather(table, b, b)

            def group(g, _, table=table):
                j0 = g * RING
                for b in range(RING):
                    wait_gather(table, b)
                    scatter(j0 + b, b)
                for b in range(RING):
                    wait_scatter(b)
                    gather(table, j0 + RING + b, b)
                return 0

            lax.fori_loop(0, NGC - 1, group, 0)
            j0 = (NGC - 1) * RING
            for b in range(RING):
                wait_gather(table, b)
                scatter(j0 + b, b)
            for b in range(RING):
                wait_scatter(b)
            plsc.subcore_barrier()
            pltpu.sync_copy(acc.at[pl.ds(row0, RPT)],
                            out_hbm.at[c, r, pl.ds(row0, RPT)])

    return body(*tabs, src_all, dst_all, ztile)


def _norm_from_deg(degsum):
    return jnp.where(degsum > 0, lax.rsqrt(jnp.maximum(degsum, 1.0)), 0.0)


def _tc_norms_scale(deg_part, x_p):
    def body(deg_ref, x_ref, ns_ref, nd_ref, *touts):
        deg = deg_ref[...]
        norm = _norm_from_deg(deg[0] + deg[1])
        ns_ref[...] = norm[:R]
        nd_ref[...] = norm[R:]
        xv = x_ref[...]
        for r in range(R):
            touts[r][...] = (xv * norm[r][:, None]).astype(jnp.bfloat16)

    grid = (NP // BLK,)
    return pl.pallas_call(
        body,
        grid=grid,
        in_specs=[
            pl.BlockSpec((NC, 2 * R, BLK), lambda i: (0, 0, i)),
            pl.BlockSpec((BLK, F), lambda i: (i, 0)),
        ],
        out_specs=[
            pl.BlockSpec((R, BLK), lambda i: (0, i)),
            pl.BlockSpec((R, BLK), lambda i: (0, i)),
        ] + [pl.BlockSpec((BLK, F), lambda i: (i, 0)) for _ in range(R)],
        out_shape=[
            jax.ShapeDtypeStruct((R, NP), jnp.float32),
            jax.ShapeDtypeStruct((R, NP), jnp.float32),
        ] + [jax.ShapeDtypeStruct((NP, F), jnp.bfloat16) for _ in range(R)],
    )(deg_part, x_p)


def _agg_rows(p, nd, w_ref, b):
    t = jnp.zeros((BLK, F), jnp.float32)
    for r in range(R):
        aggr = (p[0, r].astype(jnp.float32) + p[1, r].astype(jnp.float32))
        t = t + nd[r][:, None] * jnp.dot(
            aggr, w_ref[r], precision=lax.Precision.HIGHEST,
            preferred_element_type=jnp.float32)
    return t + (b[0] + b[1] + b[2])[None, :]


def _tc_mid(part, Ws, bs, nd, ns):
    def body(p_ref, w_ref, b_ref, nd_ref, ns_ref, *houts):
        p = p_ref[...]
        h = jnp.maximum(_agg_rows(p, nd_ref[...], w_ref, b_ref[...]), 0.0)
        ns = ns_ref[...]
        for r in range(R):
            houts[r][...] = (h * ns[r][:, None]).astype(jnp.bfloat16)

    grid = (NP // BLK,)
    return pl.pallas_call(
        body,
        grid=grid,
        in_specs=[
            pl.BlockSpec((NC, R, BLK, F), lambda i: (0, 0, i, 0)),
            pl.BlockSpec((R, F, F), lambda i: (0, 0, 0)),
            pl.BlockSpec((R, F), lambda i: (0, 0)),
            pl.BlockSpec((R, BLK), lambda i: (0, i)),
            pl.BlockSpec((R, BLK), lambda i: (0, i)),
        ],
        out_specs=[pl.BlockSpec((BLK, F), lambda i: (i, 0))
                   for _ in range(R)],
        out_shape=[jax.ShapeDtypeStruct((NP, F), jnp.bfloat16)
                   for _ in range(R)],
    )(part, Ws, bs, nd, ns)


def _tc_final(part, Ws, bs, nd):
    def body(p_ref, w_ref, b_ref, nd_ref, o_ref):
        o_ref[...] = _agg_rows(p_ref[...], nd_ref[...], w_ref, b_ref[...])

    grid = (NP // BLK,)
    return pl.pallas_call(
        body,
        grid=grid,
        in_specs=[
            pl.BlockSpec((NC, R, BLK, F), lambda i: (0, 0, i, 0)),
            pl.BlockSpec((R, F, F), lambda i: (0, 0, 0)),
            pl.BlockSpec((R, F), lambda i: (0, 0)),
            pl.BlockSpec((R, BLK), lambda i: (0, i)),
        ],
        out_specs=pl.BlockSpec((BLK, F), lambda i: (i, 0)),
        out_shape=jax.ShapeDtypeStruct((NP, F), jnp.float32),
    )(part, Ws, bs, nd)


def _prep_edges(ei):
    pad = EP - E
    fill = N + (jnp.arange(pad, dtype=jnp.int32) % (NP - N))
    src = jnp.concatenate([ei[0], fill])
    dst = jnp.concatenate([ei[1], fill])
    return src.reshape(NS, NCHUNK, CH), dst.reshape(NS, NCHUNK, CH)


def kernel(x, edge_index_rel0, edge_index_rel1, edge_index_rel2,
           W1_rel0, b1_rel0, W1_rel1, b1_rel1, W1_rel2, b1_rel2,
           W2_rel0, b2_rel0, W2_rel1, b2_rel1, W2_rel2, b2_rel2):
    s0, d0 = _prep_edges(edge_index_rel0)
    s1, d1 = _prep_edges(edge_index_rel1)
    s2, d2 = _prep_edges(edge_index_rel2)
    src_all = jnp.stack([s0, s1, s2])
    dst_all = jnp.stack([d0, d1, d2])

    x_p = jnp.zeros((NP, F), jnp.float32).at[:N].set(x)
    zrow = jnp.zeros((NP,), jnp.float32)
    ztile = jnp.zeros((CH, F), jnp.bfloat16)
    W1s = jnp.stack([W1_rel0, W1_rel1, W1_rel2])
    b1s = jnp.stack([b1_rel0, b1_rel1, b1_rel2])
    W2s = jnp.stack([W2_rel0, W2_rel1, W2_rel2])
    b2s = jnp.stack([b2_rel0, b2_rel1, b2_rel2])

    deg_part = _sc_degrees(src_all, dst_all, zrow)
    ns, nd, *xtabs = _tc_norms_scale(deg_part, x_p)
    part1 = _sc_propagate(xtabs, src_all, dst_all, ztile)
    htabs = _tc_mid(part1, W1s, b1s, nd, ns)
    part2 = _sc_propagate(htabs, src_all, dst_all, ztile)
    out = _tc_final(part2, W2s, b2s, nd)
    return out[:N]
